# baseline (speedup 1.0000x reference)
import dataclasses
import functools

import jax
import jax.numpy as jnp
from jax import lax
from jax.experimental import pallas as pl
from jax.experimental.pallas import tpu as pltpu
from jax.experimental.pallas import tpu_sc as plsc

DIM = 64
MAX_T = 72
LANES = 16
NC, NS = 2, 16
NW = NC * NS
K = 4


def _transpose_emb(time_emb):
    def body(e_ref, o_ref):
        o_ref[...] = jnp.transpose(e_ref[...], (1, 0))

    return pl.pallas_call(
        body,
        out_shape=jax.ShapeDtypeStruct((DIM, MAX_T), jnp.float32),
    )(time_emb)


def _transpose_x(x):
    B, L = x.shape
    BLK = 512

    def body(x_ref, o_ref):
        o_ref[...] = jnp.transpose(x_ref[...], (1, 0))

    return pl.pallas_call(
        body,
        grid=(B // BLK,),
        in_specs=[pl.BlockSpec((BLK, L), lambda i: (i, 0))],
        out_specs=pl.BlockSpec((L, BLK), lambda i: (0, i)),
        out_shape=jax.ShapeDtypeStruct((L, B), jnp.int32),
    )(x)


def kernel(x, time_emb):
    B, L = x.shape
    SLAB = B // NW
    NB16 = SLAB // LANES

    emb_t_flat = _transpose_emb(time_emb).reshape(DIM * MAX_T)
    xt = _transpose_x(x)

    mesh = plsc.VectorSubcoreMesh(core_axis_name="c", subcore_axis_name="s")

    cp = pltpu.CompilerParams()
    if "needs_layout_passes" in pltpu.CompilerParams.__dataclass_fields__:
        cp = dataclasses.replace(cp, needs_layout_passes=False)

    @functools.partial(
        pl.kernel,
        mesh=mesh,
        compiler_params=cp,
        out_type=jax.ShapeDtypeStruct((L, DIM, B), jnp.float32),
        scratch_types=[
            pltpu.VMEM((DIM * MAX_T,), jnp.float32),
            pltpu.VMEM((L, SLAB), jnp.int32),
            pltpu.VMEM((K, DIM, SLAB), jnp.float32),
        ]
        + [pltpu.SemaphoreType.DMA] * K,
    )
    def k(tab_hbm, xt_hbm, out_hbm, tab_v, raw_v, plane_v, *osems):
        wid = lax.axis_index("s") * NC + lax.axis_index("c")
        b0 = wid * SLAB

        pltpu.sync_copy(tab_hbm, tab_v)
        pltpu.sync_copy(xt_hbm.at[:, pl.ds(b0, SLAB)], raw_v)

        @pl.loop(0, L, step=K)
        def _(l0):
            for p in range(K):
                l = l0 + p

                @pl.when(l0 != 0)
                def _():
                    pltpu.make_async_copy(
                        plane_v.at[p],
                        out_hbm.at[l - K, :, pl.ds(b0, SLAB)],
                        osems[p],
                    ).wait()

                @plsc.parallel_loop(0, NB16, 1, unroll=4)
                def _(q):
                    idx16 = raw_v[l, pl.ds(q * LANES, LANES)]
                    c = jnp.minimum(jnp.maximum(idx16, 0), MAX_T - 1)

                    @plsc.parallel_loop(0, DIM, 1, unroll=16, carry=c)
                    def _(d, pos):
                        plane_v[p, d, pl.ds(q * LANES, LANES)] = (
                            plsc.load_gather(tab_v, [pos])
                        )
                        return pos + MAX_T

                pltpu.async_copy(
                    plane_v.at[p],
                    out_hbm.at[l, :, pl.ds(b0, SLAB)],
                    osems[p],
                )

        for p in range(K):
            pltpu.make_async_copy(
                plane_v.at[p],
                out_hbm.at[L - K + p, :, pl.ds(b0, SLAB)],
                osems[p],
            ).wait()

    out = k(emb_t_flat, xt)
    return jnp.transpose(out, (2, 0, 1))


# speedup vs baseline: 1.0058x; 1.0058x over previous
import dataclasses
import functools

import jax
import jax.numpy as jnp
from jax import lax
from jax.experimental import pallas as pl
from jax.experimental.pallas import tpu as pltpu
from jax.experimental.pallas import tpu_sc as plsc

DIM = 64
MAX_T = 72
LANES = 16
NC, NS = 2, 16
NW = NC * NS
K = 4


def _transpose_emb(time_emb):
    def body(e_ref, o_ref):
        o_ref[...] = jnp.transpose(e_ref[...], (1, 0))

    return pl.pallas_call(
        body,
        out_shape=jax.ShapeDtypeStruct((DIM, MAX_T), jnp.float32),
    )(time_emb)


def _transpose_x(x):
    B, L = x.shape
    BLK = 512

    def body(x_ref, o_ref):
        o_ref[...] = jnp.transpose(x_ref[...], (1, 0))

    return pl.pallas_call(
        body,
        grid=(B // BLK,),
        in_specs=[pl.BlockSpec((BLK, L), lambda i: (i, 0))],
        out_specs=pl.BlockSpec((L, BLK), lambda i: (0, i)),
        out_shape=jax.ShapeDtypeStruct((L, B), jnp.int32),
    )(x)


def kernel(x, time_emb):
    B, L = x.shape
    SLAB = B // NW
    NB16 = SLAB // LANES

    emb_t_flat = _transpose_emb(time_emb).reshape(DIM * MAX_T)
    xt = _transpose_x(x)

    mesh = plsc.VectorSubcoreMesh(core_axis_name="c", subcore_axis_name="s")

    cp = pltpu.CompilerParams()
    if "needs_layout_passes" in pltpu.CompilerParams.__dataclass_fields__:
        cp = dataclasses.replace(cp, needs_layout_passes=False)

    @functools.partial(
        pl.kernel,
        mesh=mesh,
        compiler_params=cp,
        out_type=jax.ShapeDtypeStruct((L, DIM, B), jnp.float32),
        scratch_types=[
            pltpu.VMEM((DIM * MAX_T,), jnp.float32),
            pltpu.VMEM((L, SLAB), jnp.int32),
            pltpu.VMEM((K, DIM, SLAB), jnp.float32),
        ]
        + [pltpu.SemaphoreType.DMA] * K,
    )
    def k(tab_hbm, xt_hbm, out_hbm, tab_v, raw_v, plane_v, *osems):
        wid = lax.axis_index("s") * NC + lax.axis_index("c")
        b0 = wid * SLAB

        pltpu.sync_copy(tab_hbm, tab_v)
        pltpu.sync_copy(xt_hbm.at[:, pl.ds(b0, SLAB)], raw_v)

        @pl.loop(0, L, step=K)
        def _(l0):
            for p in range(K):
                l = l0 + p

                @pl.when(l0 != 0)
                def _():
                    pltpu.make_async_copy(
                        plane_v.at[p],
                        out_hbm.at[l - K, :, pl.ds(b0, SLAB)],
                        osems[p],
                    ).wait()

                @plsc.parallel_loop(0, NB16, 1, unroll=2)
                def _(q):
                    idx16 = raw_v[l, pl.ds(q * LANES, LANES)]
                    c = jnp.minimum(jnp.maximum(idx16, 0), MAX_T - 1)

                    @plsc.parallel_loop(0, DIM, 1, unroll=16, carry=c)
                    def _(d, pos):
                        plane_v[p, d, pl.ds(q * LANES, LANES)] = (
                            plsc.load_gather(tab_v, [pos])
                        )
                        return pos + MAX_T

                pltpu.async_copy(
                    plane_v.at[p],
                    out_hbm.at[l, :, pl.ds(b0, SLAB)],
                    osems[p],
                )

        for p in range(K):
            pltpu.make_async_copy(
                plane_v.at[p],
                out_hbm.at[L - K + p, :, pl.ds(b0, SLAB)],
                osems[p],
            ).wait()

    out = k(emb_t_flat, xt)
    return jnp.transpose(out, (2, 0, 1))
